# baseline (speedup 1.0000x reference)
import functools

import jax
import jax.numpy as jnp
from jax import lax
from jax.experimental import pallas as pl
from jax.experimental.pallas import tpu as pltpu
from jax.experimental.pallas import tpu_sc as plsc


def kernel(inputs, embeddings):
    batch = inputs.shape[0]
    seq, emb = embeddings.shape
    info = plsc.get_sparse_core_info()
    num_workers = info.num_cores * info.num_subcores
    rows_per_worker = seq // num_workers

    mesh = plsc.VectorSubcoreMesh(core_axis_name="c", subcore_axis_name="s")

    @functools.partial(
        pl.kernel,
        mesh=mesh,
        out_type=jax.ShapeDtypeStruct((batch, seq, emb), embeddings.dtype),
        scratch_types=[pltpu.SemaphoreType.DMA] * batch,
    )
    def sc_broadcast(table_hbm, out_hbm, *sems):
        wid = lax.axis_index("s") * info.num_cores + lax.axis_index("c")
        rows = pl.ds(wid * rows_per_worker, rows_per_worker)
        handles = [
            pltpu.async_copy(table_hbm.at[rows], out_hbm.at[b, rows], sems[b])
            for b in range(batch)
        ]
        for h in handles:
            h.wait()

    return sc_broadcast(embeddings)


# speedup vs baseline: 51.1035x; 51.1035x over previous
import functools

import jax
import jax.numpy as jnp
from jax import lax
from jax.experimental import pallas as pl
from jax.experimental.pallas import tpu as pltpu
from jax.experimental.pallas import tpu_sc as plsc

_CHUNK_ROWS = 64


def kernel(inputs, embeddings):
    batch = inputs.shape[0]
    seq, emb = embeddings.shape
    info = plsc.get_sparse_core_info()
    num_workers = info.num_cores * info.num_subcores
    rows_per_worker = seq // num_workers
    n_chunks = rows_per_worker // _CHUNK_ROWS

    mesh = plsc.VectorSubcoreMesh(core_axis_name="c", subcore_axis_name="s")

    @functools.partial(
        pl.kernel,
        mesh=mesh,
        out_type=jax.ShapeDtypeStruct((batch, seq, emb), embeddings.dtype),
        scratch_types=[
            pltpu.VMEM((_CHUNK_ROWS, emb), embeddings.dtype),
            pltpu.VMEM((_CHUNK_ROWS, emb), embeddings.dtype),
            pltpu.SemaphoreType.DMA,
            pltpu.SemaphoreType.DMA,
            pltpu.SemaphoreType.DMA,
            pltpu.SemaphoreType.DMA,
        ],
    )
    def sc_broadcast(table_hbm, out_hbm, buf0, buf1, rsem0, rsem1, wsem0, wsem1):
        wid = lax.axis_index("s") * info.num_cores + lax.axis_index("c")
        base = wid * rows_per_worker
        bufs = (buf0, buf1)
        rsems = (rsem0, rsem1)
        wsems = (wsem0, wsem1)

        def chunk_slice(i):
            return pl.ds(base + i * _CHUNK_ROWS, _CHUNK_ROWS)

        pending_writes = [None, None]
        read_handles = [None] * n_chunks
        read_handles[0] = pltpu.async_copy(
            table_hbm.at[chunk_slice(0)], bufs[0], rsems[0]
        )
        for i in range(n_chunks):
            k = i % 2
            read_handles[i].wait()
            if i + 1 < n_chunks:
                k2 = (i + 1) % 2
                if pending_writes[k2] is not None:
                    for h in pending_writes[k2]:
                        h.wait()
                    pending_writes[k2] = None
                read_handles[i + 1] = pltpu.async_copy(
                    table_hbm.at[chunk_slice(i + 1)], bufs[k2], rsems[k2]
                )
            pending_writes[k] = [
                pltpu.async_copy(
                    bufs[k], out_hbm.at[(i + b) % batch, chunk_slice(i)], wsems[k]
                )
                for b in range(batch)
            ]
        for k in (0, 1):
            if pending_writes[k] is not None:
                for h in pending_writes[k]:
                    h.wait()

    return sc_broadcast(embeddings)


# speedup vs baseline: 51.1831x; 1.0016x over previous
import functools

import jax
from jax import lax
from jax.experimental import pallas as pl
from jax.experimental.pallas import tpu as pltpu
from jax.experimental.pallas import tpu_sc as plsc

_CHUNK_ROWS = 64


def kernel(inputs, embeddings):
    batch = inputs.shape[0]
    seq, emb = embeddings.shape
    info = plsc.get_sparse_core_info()
    num_workers = info.num_cores * info.num_subcores
    rows_per_worker = seq // num_workers
    n_chunks = rows_per_worker // _CHUNK_ROWS

    mesh = plsc.VectorSubcoreMesh(core_axis_name="c", subcore_axis_name="s")

    @functools.partial(
        pl.kernel,
        mesh=mesh,
        out_type=jax.ShapeDtypeStruct((batch, seq, emb), embeddings.dtype),
        scratch_types=[
            pltpu.VMEM((_CHUNK_ROWS, emb), embeddings.dtype),
            pltpu.VMEM((_CHUNK_ROWS, emb), embeddings.dtype),
            pltpu.SemaphoreType.DMA,
            pltpu.SemaphoreType.DMA,
            pltpu.SemaphoreType.DMA,
            pltpu.SemaphoreType.DMA,
        ],
    )
    def sc_broadcast(table_hbm, out_hbm, buf0, buf1, rsem0, rsem1, wsem0, wsem1):
        wid = lax.axis_index("s") * info.num_cores + lax.axis_index("c")
        base = wid * rows_per_worker
        bufs = (buf0, buf1)
        rsems = (rsem0, rsem1)
        wsems = (wsem0, wsem1)

        def chunk_slice(i):
            return pl.ds(base + i * _CHUNK_ROWS, _CHUNK_ROWS)

        pending_writes = [None, None]
        read_handles = [None] * n_chunks
        read_handles[0] = pltpu.async_copy(
            table_hbm.at[chunk_slice(0)], bufs[0], rsems[0]
        )
        for i in range(n_chunks):
            k = i % 2
            read_handles[i].wait()
            if i + 1 < n_chunks:
                k2 = (i + 1) % 2
                if pending_writes[k2] is not None:
                    for h in pending_writes[k2]:
                        h.wait()
                    pending_writes[k2] = None
                read_handles[i + 1] = pltpu.async_copy(
                    table_hbm.at[chunk_slice(i + 1)], bufs[k2], rsems[k2]
                )
            pending_writes[k] = [
                pltpu.async_copy(
                    bufs[k], out_hbm.at[(i + b) % batch, chunk_slice(i)], wsems[k]
                )
                for b in range(batch)
            ]
        for k in (0, 1):
            if pending_writes[k] is not None:
                for h in pending_writes[k]:
                    h.wait()

    return sc_broadcast(embeddings)
